# baseline (speedup 1.0000x reference)
import functools

import jax
import jax.numpy as jnp
from jax import lax
from jax.experimental import pallas as pl
from jax.experimental.pallas import tpu as pltpu
from jax.experimental.pallas import tpu_sc as plsc

B = 1024
NC, NS = 2, 16
NW = NC * NS
CHUNK_PX = 8192


def _hist_kernel(C, Pn):
    hsize = C * 2 * B
    px_per_w = Pn // NW
    chunks_per_c = px_per_w // CHUNK_PX
    nsteps = C * chunks_per_c
    bf = jnp.float32(B)

    mesh = plsc.VectorSubcoreMesh(
        core_axis_name="c", subcore_axis_name="s", num_cores=NC, num_subcores=NS
    )

    @functools.partial(
        pl.kernel,
        mesh=mesh,
        compiler_params=pltpu.CompilerParams(needs_layout_passes=False),
        out_type=jax.ShapeDtypeStruct((NW, hsize), jnp.float32),
        scratch_types=[
            pltpu.VMEM((CHUNK_PX,), jnp.float32),
            pltpu.VMEM((CHUNK_PX,), jnp.float32),
            pltpu.VMEM((CHUNK_PX,), jnp.int32),
            pltpu.VMEM((CHUNK_PX,), jnp.int32),
            pltpu.VMEM((hsize,), jnp.float32),
            pltpu.SemaphoreType.DMA,
            pltpu.SemaphoreType.DMA,
            pltpu.SemaphoreType.DMA,
            pltpu.SemaphoreType.DMA,
        ],
    )
    def body(probas_hbm, labels_hbm, out_hbm,
             pbuf0, pbuf1, lbuf0, lbuf1, hist, sp0, sp1, sl0, sl1):
        pbufs = (pbuf0, pbuf1)
        lbufs = (lbuf0, lbuf1)
        psems = (sp0, sp1)
        lsems = (sl0, sl1)

        wid = lax.axis_index("s") * NC + lax.axis_index("c")
        base = wid * px_per_w

        zeros16 = jnp.zeros((16,), jnp.float32)

        @plsc.parallel_loop(0, hsize // 16)
        def _(i):
            hist[pl.ds(i * 16, 16)] = zeros16

        def offset(step):
            c = step // chunks_per_c
            ch = step - c * chunks_per_c
            return c * Pn + base + ch * CHUNK_PX, c

        def start(step, slot):
            off, _ = offset(step)
            pltpu.async_copy(probas_hbm.at[pl.ds(off, CHUNK_PX)],
                             pbufs[slot], psems[slot])
            pltpu.async_copy(labels_hbm.at[pl.ds(off, CHUNK_PX)],
                             lbufs[slot], lsems[slot])

        def wait(slot):
            pltpu.make_async_copy(probas_hbm.at[pl.ds(0, CHUNK_PX)],
                                  pbufs[slot], psems[slot]).wait()
            pltpu.make_async_copy(labels_hbm.at[pl.ds(0, CHUNK_PX)],
                                  lbufs[slot], lsems[slot]).wait()

        ones = jnp.full((16,), 1.0, jnp.float32)

        def compute(step, slot):
            _, c = offset(step)
            coff = c * (2 * B)
            pb = pbufs[slot]
            lb = lbufs[slot]

            @plsc.parallel_loop(0, CHUNK_PX // 16, unroll=8)
            def _(i):
                o = i * 16
                vp = pb[pl.ds(o, 16)]
                vl = lb[pl.ds(o, 16)]
                fg = vl.astype(jnp.float32)
                e = jnp.abs(fg - vp)
                bi = jnp.minimum((e * bf).astype(jnp.int32), B - 1)
                idx = vl * B + bi + coff
                plsc.addupdate_scatter(hist, [idx], ones)

        start(0, 0)

        def pair(j, carry):
            s0 = 2 * j
            start(s0 + 1, 1)
            wait(0)
            compute(s0, 0)

            @pl.when(j < nsteps // 2 - 1)
            def _():
                start(s0 + 2, 0)

            wait(1)
            compute(s0 + 1, 1)
            return carry

        lax.fori_loop(0, nsteps // 2, pair, 0)

        pltpu.sync_copy(hist, out_hbm.at[wid])

    return body


def _sweep_kernel(C):
    def body(h_ref, o_ref):
        H = jnp.sum(h_ref[...], axis=0)
        m = H[:, :B] + H[:, B:]
        p = H[:, B:]
        r = lax.broadcasted_iota(jnp.int32, (B, B), 0)
        c = lax.broadcasted_iota(jnp.int32, (B, B), 1)
        tri = (r <= c).astype(jnp.float32)
        Sm = jnp.dot(m, tri, preferred_element_type=jnp.float32)
        Sp = jnp.dot(p, tri, preferred_element_type=jnp.float32)
        Mtot = Sm[:, B - 1:B]
        G = Sp[:, B - 1:B]
        Ns = Mtot - Sm
        Ne = Ns + m
        Ks = G - Sp
        Ke = Ks + p
        Js = 1.0 - (G - Ks) / jnp.maximum(G + Ns - Ks, 1.0)
        Je = 1.0 - (G - Ke) / jnp.maximum(G + Ne - Ke, 1.0)
        emid = (lax.broadcasted_iota(jnp.int32, (C, B), 1).astype(jnp.float32)
                + 0.5) * (1.0 / B)
        losses = jnp.sum(emid * (Je - Js), axis=1, keepdims=True)
        pres = (G > 0).astype(jnp.float32)
        num = jnp.sum(losses * pres)
        den = jnp.maximum(jnp.sum(pres), 1.0)
        o_ref[...] = (num / den)[None, None]

    return body


def kernel(probas, labels):
    Pn, C = probas.shape

    pf = probas.T.reshape(-1)
    lf = labels.T.reshape(-1)

    hist = _hist_kernel(C, Pn)(pf, lf)

    out = pl.pallas_call(
        _sweep_kernel(C),
        out_shape=jax.ShapeDtypeStruct((1, 1), jnp.float32),
    )(hist.reshape(NW, C, 2 * B))
    return out[0, 0]


# speedup vs baseline: 19.8676x; 19.8676x over previous
import functools

import jax
import jax.numpy as jnp
from jax import lax
from jax.experimental import pallas as pl
from jax.experimental.pallas import tpu as pltpu
from jax.experimental.pallas import tpu_sc as plsc

B = 1024
NC, NS = 2, 16
NW = NC * NS
CHUNK_PX = 8192
NBUF = 4


def _hist_kernel(C, Pn):
    hsize = C * 2 * B
    px_per_w = Pn // NW
    chunks_per_c = px_per_w // CHUNK_PX
    nsteps = C * chunks_per_c
    bf = jnp.float32(B)

    mesh = plsc.VectorSubcoreMesh(
        core_axis_name="c", subcore_axis_name="s", num_cores=NC, num_subcores=NS
    )

    @functools.partial(
        pl.kernel,
        mesh=mesh,
        compiler_params=pltpu.CompilerParams(needs_layout_passes=False),
        out_type=jax.ShapeDtypeStruct((NW, hsize), jnp.float32),
        scratch_types=(
            [pltpu.VMEM((1, CHUNK_PX), jnp.float32) for _ in range(NBUF)]
            + [pltpu.VMEM((1, CHUNK_PX), jnp.int32) for _ in range(NBUF)]
            + [pltpu.VMEM((hsize,), jnp.float32)]
            + [pltpu.SemaphoreType.DMA for _ in range(2 * NBUF)]
        ),
    )
    def body(probas_hbm, labels_hbm, out_hbm, *refs):
        pbufs = refs[0:NBUF]
        lbufs = refs[NBUF:2 * NBUF]
        hist = refs[2 * NBUF]
        psems = refs[2 * NBUF + 1:2 * NBUF + 1 + NBUF]
        lsems = refs[2 * NBUF + 1 + NBUF:2 * NBUF + 1 + 2 * NBUF]

        wid = lax.axis_index("s") * NC + lax.axis_index("c")
        base = wid * px_per_w

        zeros16 = jnp.zeros((16,), jnp.float32)

        @plsc.parallel_loop(0, hsize // 16)
        def _(i):
            hist[pl.ds(i * 16, 16)] = zeros16

        def offset(step):
            c = step // chunks_per_c
            ch = step - c * chunks_per_c
            return base + ch * CHUNK_PX, c

        def start(step, slot):
            px0, c = offset(step)
            pltpu.async_copy(probas_hbm.at[pl.ds(c, 1), pl.ds(px0, CHUNK_PX)],
                             pbufs[slot], psems[slot])
            pltpu.async_copy(labels_hbm.at[pl.ds(c, 1), pl.ds(px0, CHUNK_PX)],
                             lbufs[slot], lsems[slot])

        def wait(slot):
            pltpu.make_async_copy(probas_hbm.at[pl.ds(0, 1), pl.ds(0, CHUNK_PX)],
                                  pbufs[slot], psems[slot]).wait()
            pltpu.make_async_copy(labels_hbm.at[pl.ds(0, 1), pl.ds(0, CHUNK_PX)],
                                  lbufs[slot], lsems[slot]).wait()

        ones = jnp.full((16,), 1.0, jnp.float32)

        def compute(step, slot):
            _, c = offset(step)
            hview = hist.at[pl.ds(c * (2 * B), 2 * B)]
            pb = pbufs[slot]
            lb = lbufs[slot]

            @plsc.parallel_loop(0, CHUNK_PX // 16, unroll=16)
            def _(i):
                o = i * 16
                vp = pb[0, pl.ds(o, 16)]
                vl = lb[0, pl.ds(o, 16)]
                fg = vl.astype(jnp.float32)
                e = jnp.abs(fg - vp)
                u = jnp.minimum((e + fg) * bf, 2.0 * B - 1.0)
                plsc.addupdate_scatter(hview, [u.astype(jnp.int32)], ones)

        for k in range(NBUF - 1):
            start(k, k)

        def ring(j, carry):
            s0 = NBUF * j
            for k in range(NBUF):

                @pl.when(s0 + k + NBUF - 1 < nsteps)
                def _():
                    start(s0 + k + NBUF - 1, (k + NBUF - 1) % NBUF)

                wait(k)
                compute(s0 + k, k)
            return carry

        lax.fori_loop(0, nsteps // NBUF, ring, 0)

        pltpu.sync_copy(hist, out_hbm.at[wid])

    return body


def _sweep_kernel(C):
    def body(h_ref, o_ref):
        Hs = jnp.sum(h_ref[...], axis=0, keepdims=True)
        m_rows = []
        p_rows = []
        for cc in range(C):
            lo = Hs[:, cc * 2 * B:cc * 2 * B + B]
            hi = Hs[:, cc * 2 * B + B:(cc + 1) * 2 * B]
            m_rows.append(lo + hi)
            p_rows.append(hi)
        m = jnp.concatenate(m_rows, axis=0)
        p = jnp.concatenate(p_rows, axis=0)
        r = lax.broadcasted_iota(jnp.int32, (B, B), 0)
        c = lax.broadcasted_iota(jnp.int32, (B, B), 1)
        tri = (r <= c).astype(jnp.float32)
        Sm = jnp.dot(m, tri, preferred_element_type=jnp.float32)
        Sp = jnp.dot(p, tri, preferred_element_type=jnp.float32)
        Mtot = Sm[:, B - 1:B]
        G = Sp[:, B - 1:B]
        Ns = Mtot - Sm
        Ne = Ns + m
        Ks = G - Sp
        Ke = Ks + p
        Js = 1.0 - (G - Ks) / jnp.maximum(G + Ns - Ks, 1.0)
        Je = 1.0 - (G - Ke) / jnp.maximum(G + Ne - Ke, 1.0)
        emid = (lax.broadcasted_iota(jnp.int32, (C, B), 1).astype(jnp.float32)
                + 0.5) * (1.0 / B)
        losses = jnp.sum(emid * (Je - Js), axis=1, keepdims=True)
        pres = (G > 0).astype(jnp.float32)
        num = jnp.sum(losses * pres)
        den = jnp.maximum(jnp.sum(pres), 1.0)
        o_ref[...] = (num / den)[None, None]

    return body


def kernel(probas, labels):
    Pn, C = probas.shape

    pf = probas.T
    lf = labels.T

    hist = _hist_kernel(C, Pn)(pf, lf)

    out = pl.pallas_call(
        _sweep_kernel(C),
        out_shape=jax.ShapeDtypeStruct((1, 1), jnp.float32),
    )(hist)
    return out[0, 0]
